# baseline (speedup 1.0000x reference)
import functools

import jax
import jax.numpy as jnp
from jax import lax
from jax.experimental import pallas as pl
from jax.experimental.pallas import tpu as pltpu
from jax.experimental.pallas import tpu_sc as plsc

_NC = 2
_NS = 16
_NW = _NC * _NS


@functools.partial(jax.jit, static_argnames=("vb",))
def _transpose_call(mat_t, *, vb):
    D, V = mat_t.shape

    def xpose_kernel(in_ref, out_ref):
        out_ref[...] = in_ref[...].T

    return pl.pallas_call(
        xpose_kernel,
        grid=(pl.cdiv(V, vb),),
        in_specs=[pl.BlockSpec((D, vb), lambda i: (0, i))],
        out_specs=pl.BlockSpec((vb, D), lambda i: (i, 0)),
        out_shape=jax.ShapeDtypeStruct((V, D), jnp.float32),
    )(mat_t)


@functools.partial(jax.jit, static_argnames=("bpw", "chunk"))
def _gather_call(idx, mat, *, bpw, chunk):
    B = idx.shape[0]
    D = mat.shape[1]
    nchunk = bpw // chunk
    mesh = plsc.VectorSubcoreMesh(core_axis_name="c", subcore_axis_name="s")

    @functools.partial(
        pl.kernel,
        out_type=jax.ShapeDtypeStruct((B, D), jnp.float32),
        mesh=mesh,
        scratch_types=[
            pltpu.VMEM((bpw,), jnp.int32),
            pltpu.VMEM((2, chunk, D), jnp.float32),
            pltpu.SemaphoreType.DMA,
            pltpu.SemaphoreType.DMA,
            pltpu.SemaphoreType.DMA,
            pltpu.SemaphoreType.DMA,
        ],
        compiler_params=pltpu.CompilerParams(use_tc_tiling_on_sc=False),
    )
    def gather_kernel(idx_hbm, mat_hbm, out_hbm, idx_v, rows_v, sg0, sg1, ss0, ss1):
        wid = lax.axis_index("s") * _NC + lax.axis_index("c")
        base = wid * bpw
        pltpu.sync_copy(idx_hbm.at[pl.ds(base, bpw)], idx_v)

        sg = (sg0, sg1)
        ss = (ss0, ss1)
        gathers = [None, None]
        stores = [None, None]
        for i in range(nchunk + 1):
            if i < nchunk:
                b = i % 2
                if stores[b] is not None:
                    stores[b].wait()
                    stores[b] = None
                gathers[b] = pltpu.async_copy(
                    mat_hbm.at[idx_v.at[pl.ds(i * chunk, chunk)]],
                    rows_v.at[b],
                    sg[b],
                )
            if i >= 1:
                j = i - 1
                bj = j % 2
                gathers[bj].wait()
                stores[bj] = pltpu.async_copy(
                    rows_v.at[bj],
                    out_hbm.at[pl.ds(base + j * chunk, chunk)],
                    ss[bj],
                )
        for b in range(2):
            if stores[b] is not None:
                stores[b].wait()

    return gather_kernel(idx, mat)


def _pick_chunk(bpw, d):
    budget = 430 * 1024 - bpw * 4
    best = 8
    c = 8
    while c <= bpw:
        if bpw % c == 0 and 2 * c * d * 4 <= budget:
            best = c
        c += 8
    return best


def _pick_vb(v):
    del v
    return 2048


def kernel(x, mat):
    idx = x.reshape(-1)
    B = idx.shape[0]
    D = mat.shape[1]
    bpw = B // _NW
    chunk = _pick_chunk(bpw, D)
    mat_lin = _transpose_call(mat.T, vb=_pick_vb(mat.shape[0]))
    return _gather_call(idx, mat_lin, bpw=bpw, chunk=chunk)
